# baseline (speedup 1.0000x reference)
import functools

import jax
import jax.numpy as jnp
from jax.experimental import pallas as pl
from jax.experimental.pallas import tpu as pltpu
from jax.experimental.pallas import tpu_sc as plsc

EPS = 1.1920929e-07
NEG = -1e30
DH = 64


def _rms(x, w):
    return x * jax.lax.rsqrt(jnp.mean(x * x, axis=-1, keepdims=True) + EPS) * w


def _dot(a, b, dims, out=jnp.float32):
    return jax.lax.dot_general(a, b, (dims, ((), ())),
                               preferred_element_type=out)


def _cumsum_lanes(x):
    S = x.shape[1]
    lane = jax.lax.broadcasted_iota(jnp.int32, x.shape, 1)
    sh = 1
    while sh < S:
        r = pltpu.roll(x, sh, axis=1)
        x = x + jnp.where(lane >= sh, r, 0.0)
        sh *= 2
    return x


def _router_kernel(K, KP, x_ref, gw_ref, gb_ref, idxf_ref, idxi_ref, aux_ref):
    S = x_ref.shape[0]
    l = _dot(gw_ref[...], x_ref[...], ((1,), (1,))) + gb_ref[...]
    p = jax.nn.sigmoid(l)
    pm = jnp.sum(p, axis=1, keepdims=True) / S
    sp = (pm - 0.1) ** 2
    ent = -(p * jnp.log(p + 1e-10) + (1.0 - p) * jnp.log(1.0 - p + 1e-10))
    aux_ref[...] = 0.1 * sp + 0.01 * (jnp.sum(ent, axis=1, keepdims=True) / S)
    u = jax.lax.bitcast_convert_type(l, jnp.int32)
    key = jnp.where(u >= 0, u, u ^ jnp.int32(0x7FFFFFFF))
    MIN32 = jnp.int32(-(2 ** 31))
    tb = jnp.zeros((1, 1), jnp.int32)
    for b in range(31, -1, -1):
        bit = MIN32 if b == 31 else jnp.int32(1 << b)
        cand = tb | bit
        thr = cand ^ MIN32
        cnt = jnp.sum(jnp.where(key >= thr, 1.0, 0.0), axis=1, keepdims=True)
        tb = jnp.where(cnt >= K, cand, tb)
    vk = tb ^ MIN32
    gt = key > vk
    eq = key == vk
    C = jnp.sum(jnp.where(gt, 1.0, 0.0), axis=1, keepdims=True)
    eqf = jnp.where(eq, 1.0, 0.0)
    eqpos = _cumsum_lanes(eqf) - eqf
    sel = jnp.where(gt, 1.0, jnp.where(eq & (eqpos < (K - C)), 1.0, 0.0))
    pos = _cumsum_lanes(sel) - sel
    rows = jax.lax.broadcasted_iota(jnp.int32, (KP, 1), 0).astype(jnp.float32)
    lane = jax.lax.broadcasted_iota(jnp.int32, (KP, S), 1).astype(jnp.float32)
    onehot = jnp.where((sel > 0.5) & (pos == rows), 1.0, 0.0)
    idxf = jnp.sum(onehot * lane, axis=1, keepdims=True)
    idxf_ref[...] = idxf
    idxi_ref[...] = idxf.astype(jnp.int32)


def _kv_kernel(KVH, x_ref, anw_ref, wk_ref, wkr_ref, wv_ref, cos_ref, sin_ref,
               k_ref, v_ref):
    xn = _rms(x_ref[...], anw_ref[...])
    k0 = _dot(xn, wk_ref[...], ((1,), (1,)))
    kr = _dot(xn, wkr_ref[...], ((1,), (1,)))
    cos = jnp.concatenate([cos_ref[...]] * KVH, axis=1)
    sin = jnp.concatenate([sin_ref[...]] * KVH, axis=1)
    k = (k0 * cos + kr * sin).astype(jnp.bfloat16)
    v = _dot(xn, wv_ref[...], ((1,), (1,))).astype(jnp.bfloat16)
    for g in range(KVH):
        k_ref[g] = k[:, g * DH:(g + 1) * DH]
        v_ref[g] = v[:, g * DH:(g + 1) * DH]


def _make_sc_gather(D, CS, KP):
    info = plsc.get_sparse_core_info()
    NC, NS = info.num_cores, info.num_subcores
    NW = NC * NS
    BPW = KP // NW
    mesh = plsc.VectorSubcoreMesh(core_axis_name="c", subcore_axis_name="s")

    @functools.partial(
        pl.kernel, mesh=mesh,
        out_type=[jax.ShapeDtypeStruct((KP, D), jnp.float32),
                  jax.ShapeDtypeStruct((KP, CS), jnp.float32)],
        scratch_types=[pltpu.VMEM((BPW,), jnp.int32),
                       pltpu.VMEM((BPW, D), jnp.float32),
                       pltpu.VMEM((BPW, CS), jnp.float32),
                       pltpu.SemaphoreType.DMA],
    )
    def gather(x_hbm, cs_hbm, idx_hbm, xo_hbm, co_hbm, idx_v, xrows, crows, sem):
        wid = jax.lax.axis_index("s") * NC + jax.lax.axis_index("c")
        base = wid * BPW
        pltpu.sync_copy(idx_hbm.at[pl.ds(base, BPW)], idx_v)
        pltpu.async_copy(x_hbm.at[idx_v], xrows, sem).wait()
        pltpu.async_copy(cs_hbm.at[idx_v], crows, sem).wait()
        pltpu.sync_copy(xrows, xo_hbm.at[pl.ds(base, BPW)])
        pltpu.sync_copy(crows, co_hbm.at[pl.ds(base, BPW)])

    return gather


def _qsel_kernel(H, xs_ref, c_ref, anw_ref, wq_ref, wqr_ref, q_ref):
    xn = _rms(xs_ref[...], anw_ref[...])
    q0 = _dot(xn, wq_ref[...], ((1,), (1,)))
    qr = _dot(xn, wqr_ref[...], ((1,), (1,)))
    cos = jnp.concatenate([c_ref[:, :DH]] * H, axis=1)
    sin = jnp.concatenate([c_ref[:, DH:]] * H, axis=1)
    q = (q0 * cos + qr * sin).astype(jnp.bfloat16)
    for h in range(H):
        q_ref[h] = q[:, h * DH:(h + 1) * DH]


def _attn_kernel(QB, KT, K, scale, q_ref, k_ref, v_ref, idxf_ref, idxi_ref, o_ref):
    qb = pl.program_id(1)
    last_row = jnp.minimum((qb + 1) * QB - 1, K - 1)
    nt = idxi_ref[last_row, 0] // KT + 1
    q = (q_ref[0].astype(jnp.float32) * scale).astype(jnp.bfloat16)
    idxc = idxf_ref[...]
    jc = jax.lax.broadcasted_iota(jnp.int32, (QB, KT), 1).astype(jnp.float32)

    def body(t, carry):
        m, den, acc = carry
        k = k_ref[0, pl.ds(t * KT, KT), :]
        v = v_ref[0, pl.ds(t * KT, KT), :]
        s = _dot(q, k, ((1,), (1,)))
        s = jnp.where(idxc - (t * KT).astype(jnp.float32) >= jc, s, NEG)
        mt = jnp.maximum(m, jnp.max(s, axis=1, keepdims=True))
        alpha = jnp.exp(m - mt)
        p = jnp.exp(s - mt)
        den = den * alpha + jnp.sum(p, axis=1, keepdims=True)
        acc = acc * alpha + _dot(p.astype(jnp.bfloat16), v, ((1,), (0,)))
        return mt, den, acc

    m0 = jnp.full((QB, 1), NEG, jnp.float32)
    z1 = jnp.zeros((QB, 1), jnp.float32)
    z2 = jnp.zeros((QB, DH), jnp.float32)
    m, den, acc = jax.lax.fori_loop(0, nt, body, (m0, z1, z2))
    o_ref[0] = acc / den


def _oscatter_kernel(H, K, NSB, SB, x_ref, o_ref, wo_ref, idxi_ref,
                     h_ref, hf_ref, hl_ref, op_ref):
    o = jnp.concatenate([o_ref[h] for h in range(H)], axis=1)
    op_ref[...] = _dot(o, wo_ref[...], ((1,), (1,)))
    h_ref[...] = x_ref[...]

    def scat(i, _):
        r = idxi_ref[i, 0]
        h_ref[pl.ds(r, 1), :] = h_ref[pl.ds(r, 1), :] + op_ref[pl.ds(i, 1), :]
        return 0
    jax.lax.fori_loop(0, K, scat, 0)
    for s in range(NSB):
        hf_ref[s, 0:1, :] = h_ref[s * SB:s * SB + 1, :]
        hl_ref[s, 0:1, :] = h_ref[(s + 1) * SB - 1:(s + 1) * SB, :]


def _mlp_kernel(NSB, rnw_ref, w0_ref, w1_ref, w2_ref, h_ref, hl_ref, hf_ref,
                wg_ref, wu_ref, wd_ref, y_ref):
    s = pl.program_id(0)
    rnw = rnw_ref[...]
    h = h_ref[...]
    hn = _rms(h, rnw)
    prev_row = jnp.where(s > 0, _rms(hl_ref[0], rnw), 0.0)
    next_row = jnp.where(s < NSB - 1, _rms(hf_ref[0], rnw), 0.0)
    prev = jnp.concatenate([prev_row, hn[:-1, :]], axis=0)
    nxt = jnp.concatenate([hn[1:, :], next_row], axis=0)
    hc = (w0_ref[...] * prev + w1_ref[...] * hn + w2_ref[...] * nxt).astype(jnp.float8_e4m3fn)
    g = _dot(hc, wg_ref[...], ((1,), (1,)))
    u = _dot(hc, wu_ref[...], ((1,), (1,)))
    a = (g * jax.nn.sigmoid(g) * u).astype(jnp.float8_e4m3fn)
    y_ref[...] = h + _dot(a, wd_ref[...], ((1,), (1,)))


def _rope_rows(W, nheads):
    Wh = W.reshape(nheads, DH, -1)
    return jnp.concatenate([-Wh[:, DH // 2:], Wh[:, :DH // 2]], axis=1).reshape(W.shape)


def kernel(x, Wq, Wk, Wv, Wo, attn_norm_w, gate_w, gate_b, log_temp,
           r_norm_w, conv_w, Wg, Wu, Wd):
    B, S, D = x.shape
    H = Wq.shape[0] // DH
    KVH = Wk.shape[0] // DH
    HID = Wg.shape[0]
    KQ = max(1, int(S * 0.1))
    KP = ((KQ + 127) // 128) * 128
    SB = min(512, S)
    NSB = S // SB
    GRP = H // KVH

    x2 = x.reshape(S, D)
    f32 = jnp.float32
    bf16 = jnp.bfloat16

    inv_freq = 1.0 / (10000.0 ** (jnp.arange(0, DH, 2, dtype=f32) / DH))
    fr = jnp.outer(jnp.arange(S, dtype=f32), inv_freq)
    emb = jnp.concatenate([fr, fr], axis=-1)
    cos64 = jnp.cos(emb)
    sin64 = jnp.sin(emb)
    cossin = jnp.concatenate([cos64, sin64], axis=1)
    WqR = _rope_rows(Wq, H)
    WkR = _rope_rows(Wk, KVH)
    anw = attn_norm_w.reshape(1, D)
    rnw = r_norm_w.reshape(1, D)
    gw = gate_w.reshape(1, D)
    gb = gate_b.reshape(1, 1)
    w0 = conv_w[:, 0, 0].reshape(1, D)
    w1 = conv_w[:, 0, 1].reshape(1, D)
    w2 = conv_w[:, 0, 2].reshape(1, D)

    full = lambda shp: pl.BlockSpec(shp, lambda *_: tuple(0 for _ in shp))
    smem = pl.BlockSpec(memory_space=pltpu.SMEM)

    idx_f, idx_i, aux = pl.pallas_call(
        functools.partial(_router_kernel, KQ, KP),
        out_shape=[jax.ShapeDtypeStruct((KP, 1), f32),
                   jax.ShapeDtypeStruct((KP, 1), jnp.int32),
                   jax.ShapeDtypeStruct((1, 1), f32)],
        in_specs=[full((S, D)), full((1, D)), full((1, 1))],
        out_specs=[full((KP, 1)), full((KP, 1)), full((1, 1))],
    )(x2, gw, gb)

    k4, v4 = pl.pallas_call(
        functools.partial(_kv_kernel, KVH),
        grid=(NSB,),
        out_shape=[jax.ShapeDtypeStruct((KVH, S, DH), bf16),
                   jax.ShapeDtypeStruct((KVH, S, DH), bf16)],
        in_specs=[pl.BlockSpec((SB, D), lambda s: (s, 0)),
                  full((1, D)),
                  full((KVH * DH, D)), full((KVH * DH, D)), full((KVH * DH, D)),
                  pl.BlockSpec((SB, DH), lambda s: (s, 0)),
                  pl.BlockSpec((SB, DH), lambda s: (s, 0))],
        out_specs=[pl.BlockSpec((KVH, SB, DH), lambda s: (0, s, 0)),
                   pl.BlockSpec((KVH, SB, DH), lambda s: (0, s, 0))],
    )(x2, anw, Wk, WkR, Wv, cos64, sin64)

    xsel, cssel = _make_sc_gather(D, 2 * DH, KP)(x2, cossin, idx_i.reshape(KP))

    q3 = pl.pallas_call(
        functools.partial(_qsel_kernel, H),
        out_shape=jax.ShapeDtypeStruct((H, KP, DH), bf16),
        in_specs=[full((KP, D)), full((KP, 2 * DH)), full((1, D)),
                  full((D, D)), full((D, D))],
        out_specs=full((H, KP, DH)),
    )(xsel, cssel, anw, Wq, WqR)

    QB = min(256, KP)
    KT = min(1024, S)
    NQB = KP // QB
    o3 = pl.pallas_call(
        functools.partial(_attn_kernel, QB, KT, KQ, 1.0 / (DH ** 0.5)),
        grid=(H, NQB),
        out_shape=jax.ShapeDtypeStruct((H, KP, DH), f32),
        in_specs=[pl.BlockSpec((1, QB, DH), lambda h, qb: (h, qb, 0)),
                  pl.BlockSpec((1, S, DH), lambda h, qb: (h // GRP, 0, 0)),
                  pl.BlockSpec((1, S, DH), lambda h, qb: (h // GRP, 0, 0)),
                  pl.BlockSpec((QB, 1), lambda h, qb: (qb, 0)),
                  smem],
        out_specs=pl.BlockSpec((1, QB, DH), lambda h, qb: (h, qb, 0)),
    )(q3, k4, v4, idx_f, idx_i)

    SBM = min(256, S)
    NBM = S // SBM
    h, hfirst, hlast = pl.pallas_call(
        functools.partial(_oscatter_kernel, H, KQ, NBM, SBM),
        out_shape=[jax.ShapeDtypeStruct((S, D), f32),
                   jax.ShapeDtypeStruct((NBM, 1, D), f32),
                   jax.ShapeDtypeStruct((NBM, 1, D), f32)],
        in_specs=[full((S, D)), full((H, KP, DH)), full((D, D)), smem],
        out_specs=[full((S, D)), full((NBM, 1, D)), full((NBM, 1, D))],
        scratch_shapes=[pltpu.VMEM((KP, D), f32)],
    )(x2, o3, Wo, idx_i)

    SBL = min(512, S)
    NBL = S // SBL
    RL = SBL // SBM
    y = pl.pallas_call(
        functools.partial(_mlp_kernel, NBL),
        grid=(NBL,),
        out_shape=jax.ShapeDtypeStruct((S, D), f32),
        in_specs=[full((1, D)), full((1, D)), full((1, D)), full((1, D)),
                  pl.BlockSpec((SBL, D), lambda s: (s, 0)),
                  pl.BlockSpec((1, 1, D), lambda s: (jnp.maximum(s * RL - 1, 0), 0, 0)),
                  pl.BlockSpec((1, 1, D),
                               lambda s: (jnp.minimum((s + 1) * RL, NBM - 1), 0, 0)),
                  full((HID, D)), full((HID, D)), full((D, HID))],
        out_specs=pl.BlockSpec((SBL, D), lambda s: (s, 0)),
    )(rnw, w0, w1, w2, h, hlast, hfirst, Wg.astype(jnp.float8_e4m3fn),
      Wu.astype(jnp.float8_e4m3fn), Wd.astype(jnp.float8_e4m3fn))

    return y.reshape(B, S, D), aux[0, 0]


# speedup vs baseline: 1.0282x; 1.0282x over previous
import functools

import jax
import jax.numpy as jnp
from jax.experimental import pallas as pl
from jax.experimental.pallas import tpu as pltpu
from jax.experimental.pallas import tpu_sc as plsc

EPS = 1.1920929e-07
NEG = -1e30
DH = 64


def _rms(x, w):
    return x * jax.lax.rsqrt(jnp.mean(x * x, axis=-1, keepdims=True) + EPS) * w


def _dot(a, b, dims, out=jnp.float32):
    return jax.lax.dot_general(a, b, (dims, ((), ())),
                               preferred_element_type=out)


def _cumsum_lanes(x):
    S = x.shape[1]
    lane = jax.lax.broadcasted_iota(jnp.int32, x.shape, 1)
    sh = 1
    while sh < S:
        r = pltpu.roll(x, sh, axis=1)
        x = x + jnp.where(lane >= sh, r, 0.0)
        sh *= 2
    return x


def _router_kernel(K, KP, x_ref, gw_ref, gb_ref, idxf_ref, idxi_ref, aux_ref):
    S = x_ref.shape[0]
    l = _dot(gw_ref[...], x_ref[...], ((1,), (1,))) + gb_ref[...]
    p = jax.nn.sigmoid(l)
    pm = jnp.sum(p, axis=1, keepdims=True) / S
    sp = (pm - 0.1) ** 2
    ent = -(p * jnp.log(p + 1e-10) + (1.0 - p) * jnp.log(1.0 - p + 1e-10))
    aux_ref[...] = 0.1 * sp + 0.01 * (jnp.sum(ent, axis=1, keepdims=True) / S)
    u = jax.lax.bitcast_convert_type(l, jnp.int32)
    key = jnp.where(u >= 0, u, u ^ jnp.int32(0x7FFFFFFF))
    MIN32 = jnp.int32(-(2 ** 31))
    tb = jnp.zeros((1, 1), jnp.int32)
    for b in range(31, -1, -1):
        bit = MIN32 if b == 31 else jnp.int32(1 << b)
        cand = tb | bit
        thr = cand ^ MIN32
        cnt = jnp.sum(jnp.where(key >= thr, 1.0, 0.0), axis=1, keepdims=True)
        tb = jnp.where(cnt >= K, cand, tb)
    vk = tb ^ MIN32
    gt = key > vk
    eq = key == vk
    C = jnp.sum(jnp.where(gt, 1.0, 0.0), axis=1, keepdims=True)
    eqf = jnp.where(eq, 1.0, 0.0)
    eqpos = _cumsum_lanes(eqf) - eqf
    sel = jnp.where(gt, 1.0, jnp.where(eq & (eqpos < (K - C)), 1.0, 0.0))
    pos = _cumsum_lanes(sel) - sel
    rows = jax.lax.broadcasted_iota(jnp.int32, (KP, 1), 0).astype(jnp.float32)
    lane = jax.lax.broadcasted_iota(jnp.int32, (KP, S), 1).astype(jnp.float32)
    onehot = jnp.where((sel > 0.5) & (pos == rows), 1.0, 0.0)
    idxf = jnp.sum(onehot * lane, axis=1, keepdims=True)
    idxf_ref[...] = idxf
    idxi_ref[...] = idxf.astype(jnp.int32)


def _kv_kernel(KVH, x_ref, anw_ref, wk_ref, wkr_ref, wv_ref, cos_ref, sin_ref,
               k_ref, v_ref):
    xn = _rms(x_ref[...], anw_ref[...])
    k0 = _dot(xn, wk_ref[...], ((1,), (1,)))
    kr = _dot(xn, wkr_ref[...], ((1,), (1,)))
    cos = jnp.concatenate([cos_ref[...]] * KVH, axis=1)
    sin = jnp.concatenate([sin_ref[...]] * KVH, axis=1)
    k = (k0 * cos + kr * sin).astype(jnp.bfloat16)
    v = _dot(xn, wv_ref[...], ((1,), (1,))).astype(jnp.bfloat16)
    for g in range(KVH):
        k_ref[g] = k[:, g * DH:(g + 1) * DH]
        v_ref[g] = v[:, g * DH:(g + 1) * DH]


def _make_sc_gather(D, CS, KP):
    info = plsc.get_sparse_core_info()
    NC, NS = info.num_cores, info.num_subcores
    NW = NC * NS
    BPW = KP // NW
    mesh = plsc.VectorSubcoreMesh(core_axis_name="c", subcore_axis_name="s")

    @functools.partial(
        pl.kernel, mesh=mesh,
        out_type=[jax.ShapeDtypeStruct((KP, D), jnp.float32),
                  jax.ShapeDtypeStruct((KP, CS), jnp.float32)],
        scratch_types=[pltpu.VMEM((BPW,), jnp.int32),
                       pltpu.VMEM((BPW, D), jnp.float32),
                       pltpu.VMEM((BPW, CS), jnp.float32),
                       pltpu.SemaphoreType.DMA],
    )
    def gather(x_hbm, cs_hbm, idx_hbm, xo_hbm, co_hbm, idx_v, xrows, crows, sem):
        wid = jax.lax.axis_index("s") * NC + jax.lax.axis_index("c")
        base = wid * BPW
        pltpu.sync_copy(idx_hbm.at[pl.ds(base, BPW)], idx_v)
        pltpu.async_copy(x_hbm.at[idx_v], xrows, sem).wait()
        pltpu.async_copy(cs_hbm.at[idx_v], crows, sem).wait()
        pltpu.sync_copy(xrows, xo_hbm.at[pl.ds(base, BPW)])
        pltpu.sync_copy(crows, co_hbm.at[pl.ds(base, BPW)])

    return gather


def _qsel_kernel(H, xs_ref, c_ref, anw_ref, wq_ref, wqr_ref, q_ref):
    xn = _rms(xs_ref[...], anw_ref[...])
    q0 = _dot(xn, wq_ref[...], ((1,), (1,)))
    qr = _dot(xn, wqr_ref[...], ((1,), (1,)))
    cos = jnp.concatenate([c_ref[:, :DH]] * H, axis=1)
    sin = jnp.concatenate([c_ref[:, DH:]] * H, axis=1)
    q = (q0 * cos + qr * sin).astype(jnp.bfloat16)
    for h in range(H):
        q_ref[h] = q[:, h * DH:(h + 1) * DH]


def _attn_kernel(QB, KT, K, scale, q_ref, k_ref, v_ref, idxf_ref, idxi_ref, o_ref):
    qb = pl.program_id(1)
    last_row = jnp.minimum((qb + 1) * QB - 1, K - 1)
    nt = idxi_ref[last_row, 0] // KT + 1
    q = (q_ref[0].astype(jnp.float32) * scale).astype(jnp.bfloat16)
    idxc = idxf_ref[...]
    jc = jax.lax.broadcasted_iota(jnp.int32, (QB, KT), 1).astype(jnp.float32)

    def body(t, carry):
        m, den, acc = carry
        k = k_ref[0, pl.ds(t * KT, KT), :]
        v = v_ref[0, pl.ds(t * KT, KT), :]
        s = _dot(q, k, ((1,), (1,)))
        s = jnp.where(idxc - (t * KT).astype(jnp.float32) >= jc, s, NEG)
        mt = jnp.maximum(m, jnp.max(s, axis=1, keepdims=True))
        alpha = jnp.exp(m - mt)
        p = jnp.exp(s - mt)
        den = den * alpha + jnp.sum(p, axis=1, keepdims=True)
        acc = acc * alpha + _dot(p.astype(jnp.bfloat16), v, ((1,), (0,)))
        return mt, den, acc

    m0 = jnp.full((QB, 1), NEG, jnp.float32)
    z1 = jnp.zeros((QB, 1), jnp.float32)
    z2 = jnp.zeros((QB, DH), jnp.float32)
    m, den, acc = jax.lax.fori_loop(0, nt, body, (m0, z1, z2))
    o_ref[0] = acc / den


def _oscatter_kernel(H, K, NSB, SB, x_ref, o_ref, wo_ref, idxi_ref,
                     h_ref, hf_ref, hl_ref, op_ref):
    o = jnp.concatenate([o_ref[h] for h in range(H)], axis=1)
    op_ref[...] = _dot(o, wo_ref[...], ((1,), (1,)))
    h_ref[...] = x_ref[...]

    def scat(i, _):
        r = idxi_ref[i, 0]
        h_ref[pl.ds(r, 1), :] = h_ref[pl.ds(r, 1), :] + op_ref[pl.ds(i, 1), :]
        return 0
    jax.lax.fori_loop(0, K, scat, 0)
    for s in range(NSB):
        hf_ref[s, 0:1, :] = h_ref[s * SB:s * SB + 1, :]
        hl_ref[s, 0:1, :] = h_ref[(s + 1) * SB - 1:(s + 1) * SB, :]


def _wcast_kernel(wg_ref, wu_ref, wd_ref, og_ref, ou_ref, od_ref):
    og_ref[...] = wg_ref[...].astype(jnp.float8_e4m3fn)
    ou_ref[...] = wu_ref[...].astype(jnp.float8_e4m3fn)
    od_ref[...] = wd_ref[...].astype(jnp.float8_e4m3fn)


def _mlp_kernel(NSB, rnw_ref, w0_ref, w1_ref, w2_ref, h_ref, hl_ref, hf_ref,
                wg_ref, wu_ref, wd_ref, y_ref):
    s = pl.program_id(0)
    rnw = rnw_ref[...]
    h = h_ref[...]
    hn = _rms(h, rnw)
    prev_row = jnp.where(s > 0, _rms(hl_ref[0], rnw), 0.0)
    next_row = jnp.where(s < NSB - 1, _rms(hf_ref[0], rnw), 0.0)
    prev = jnp.concatenate([prev_row, hn[:-1, :]], axis=0)
    nxt = jnp.concatenate([hn[1:, :], next_row], axis=0)
    hc = (w0_ref[...] * prev + w1_ref[...] * hn + w2_ref[...] * nxt).astype(jnp.float8_e4m3fn)
    g = _dot(hc, wg_ref[...], ((1,), (1,)))
    u = _dot(hc, wu_ref[...], ((1,), (1,)))
    a = (g * jax.nn.sigmoid(g) * u).astype(jnp.float8_e4m3fn)
    y_ref[...] = h + _dot(a, wd_ref[...], ((1,), (1,)))


def _rope_rows(W, nheads):
    Wh = W.reshape(nheads, DH, -1)
    return jnp.concatenate([-Wh[:, DH // 2:], Wh[:, :DH // 2]], axis=1).reshape(W.shape)


def kernel(x, Wq, Wk, Wv, Wo, attn_norm_w, gate_w, gate_b, log_temp,
           r_norm_w, conv_w, Wg, Wu, Wd):
    B, S, D = x.shape
    H = Wq.shape[0] // DH
    KVH = Wk.shape[0] // DH
    HID = Wg.shape[0]
    KQ = max(1, int(S * 0.1))
    KP = ((KQ + 127) // 128) * 128
    SB = min(512, S)
    NSB = S // SB
    GRP = H // KVH

    x2 = x.reshape(S, D)
    f32 = jnp.float32
    bf16 = jnp.bfloat16

    inv_freq = 1.0 / (10000.0 ** (jnp.arange(0, DH, 2, dtype=f32) / DH))
    fr = jnp.outer(jnp.arange(S, dtype=f32), inv_freq)
    emb = jnp.concatenate([fr, fr], axis=-1)
    cos64 = jnp.cos(emb)
    sin64 = jnp.sin(emb)
    cossin = jnp.concatenate([cos64, sin64], axis=1)
    WqR = _rope_rows(Wq, H)
    WkR = _rope_rows(Wk, KVH)
    anw = attn_norm_w.reshape(1, D)
    rnw = r_norm_w.reshape(1, D)
    gw = gate_w.reshape(1, D)
    gb = gate_b.reshape(1, 1)
    w0 = conv_w[:, 0, 0].reshape(1, D)
    w1 = conv_w[:, 0, 1].reshape(1, D)
    w2 = conv_w[:, 0, 2].reshape(1, D)

    full = lambda shp: pl.BlockSpec(shp, lambda *_: tuple(0 for _ in shp))
    smem = pl.BlockSpec(memory_space=pltpu.SMEM)

    idx_f, idx_i, aux = pl.pallas_call(
        functools.partial(_router_kernel, KQ, KP),
        out_shape=[jax.ShapeDtypeStruct((KP, 1), f32),
                   jax.ShapeDtypeStruct((KP, 1), jnp.int32),
                   jax.ShapeDtypeStruct((1, 1), f32)],
        in_specs=[full((S, D)), full((1, D)), full((1, 1))],
        out_specs=[full((KP, 1)), full((KP, 1)), full((1, 1))],
    )(x2, gw, gb)

    k4, v4 = pl.pallas_call(
        functools.partial(_kv_kernel, KVH),
        grid=(NSB,),
        out_shape=[jax.ShapeDtypeStruct((KVH, S, DH), bf16),
                   jax.ShapeDtypeStruct((KVH, S, DH), bf16)],
        in_specs=[pl.BlockSpec((SB, D), lambda s: (s, 0)),
                  full((1, D)),
                  full((KVH * DH, D)), full((KVH * DH, D)), full((KVH * DH, D)),
                  pl.BlockSpec((SB, DH), lambda s: (s, 0)),
                  pl.BlockSpec((SB, DH), lambda s: (s, 0))],
        out_specs=[pl.BlockSpec((KVH, SB, DH), lambda s: (0, s, 0)),
                   pl.BlockSpec((KVH, SB, DH), lambda s: (0, s, 0))],
    )(x2, anw, Wk, WkR, Wv, cos64, sin64)

    xsel, cssel = _make_sc_gather(D, 2 * DH, KP)(x2, cossin, idx_i.reshape(KP))

    q3 = pl.pallas_call(
        functools.partial(_qsel_kernel, H),
        out_shape=jax.ShapeDtypeStruct((H, KP, DH), bf16),
        in_specs=[full((KP, D)), full((KP, 2 * DH)), full((1, D)),
                  full((D, D)), full((D, D))],
        out_specs=full((H, KP, DH)),
    )(xsel, cssel, anw, Wq, WqR)

    QB = min(256, KP)
    KT = min(1024, S)
    NQB = KP // QB
    o3 = pl.pallas_call(
        functools.partial(_attn_kernel, QB, KT, KQ, 1.0 / (DH ** 0.5)),
        grid=(H, NQB),
        out_shape=jax.ShapeDtypeStruct((H, KP, DH), f32),
        in_specs=[pl.BlockSpec((1, QB, DH), lambda h, qb: (h, qb, 0)),
                  pl.BlockSpec((1, S, DH), lambda h, qb: (h // GRP, 0, 0)),
                  pl.BlockSpec((1, S, DH), lambda h, qb: (h // GRP, 0, 0)),
                  pl.BlockSpec((QB, 1), lambda h, qb: (qb, 0)),
                  smem],
        out_specs=pl.BlockSpec((1, QB, DH), lambda h, qb: (h, qb, 0)),
    )(q3, k4, v4, idx_f, idx_i)

    SBM = min(256, S)
    NBM = S // SBM
    h, hfirst, hlast = pl.pallas_call(
        functools.partial(_oscatter_kernel, H, KQ, NBM, SBM),
        out_shape=[jax.ShapeDtypeStruct((S, D), f32),
                   jax.ShapeDtypeStruct((NBM, 1, D), f32),
                   jax.ShapeDtypeStruct((NBM, 1, D), f32)],
        in_specs=[full((S, D)), full((H, KP, DH)), full((D, D)), smem],
        out_specs=[full((S, D)), full((NBM, 1, D)), full((NBM, 1, D))],
        scratch_shapes=[pltpu.VMEM((KP, D), f32)],
    )(x2, o3, Wo, idx_i)

    NW8 = 8 if HID % 8 == 0 else 1
    Wg8, Wu8, Wd8 = pl.pallas_call(
        _wcast_kernel,
        grid=(NW8,),
        out_shape=[jax.ShapeDtypeStruct((HID, D), jnp.float8_e4m3fn),
                   jax.ShapeDtypeStruct((HID, D), jnp.float8_e4m3fn),
                   jax.ShapeDtypeStruct((D, HID), jnp.float8_e4m3fn)],
        in_specs=[pl.BlockSpec((HID // NW8, D), lambda s: (s, 0)),
                  pl.BlockSpec((HID // NW8, D), lambda s: (s, 0)),
                  pl.BlockSpec((D // NW8, HID), lambda s: (s, 0))],
        out_specs=[pl.BlockSpec((HID // NW8, D), lambda s: (s, 0)),
                   pl.BlockSpec((HID // NW8, D), lambda s: (s, 0)),
                   pl.BlockSpec((D // NW8, HID), lambda s: (s, 0))],
    )(Wg, Wu, Wd)

    SBL = min(512, S)
    NBL = S // SBL
    RL = SBL // SBM
    y = pl.pallas_call(
        functools.partial(_mlp_kernel, NBL),
        grid=(NBL,),
        out_shape=jax.ShapeDtypeStruct((S, D), f32),
        in_specs=[full((1, D)), full((1, D)), full((1, D)), full((1, D)),
                  pl.BlockSpec((SBL, D), lambda s: (s, 0)),
                  pl.BlockSpec((1, 1, D), lambda s: (jnp.maximum(s * RL - 1, 0), 0, 0)),
                  pl.BlockSpec((1, 1, D),
                               lambda s: (jnp.minimum((s + 1) * RL, NBM - 1), 0, 0)),
                  full((HID, D)), full((HID, D)), full((D, HID))],
        out_specs=pl.BlockSpec((SBL, D), lambda s: (s, 0)),
    )(rnw, w0, w1, w2, h, hlast, hfirst, Wg8, Wu8, Wd8)

    return y.reshape(B, S, D), aux[0, 0]


# speedup vs baseline: 1.0706x; 1.0412x over previous
import functools

import jax
import jax.numpy as jnp
from jax.experimental import pallas as pl
from jax.experimental.pallas import tpu as pltpu
from jax.experimental.pallas import tpu_sc as plsc

EPS = 1.1920929e-07
NEG = -1e30
DH = 64


def _rms(x, w):
    return x * jax.lax.rsqrt(jnp.mean(x * x, axis=-1, keepdims=True) + EPS) * w


def _dot(a, b, dims, out=jnp.float32):
    return jax.lax.dot_general(a, b, (dims, ((), ())),
                               preferred_element_type=out)


def _cumsum_lanes(x):
    S = x.shape[1]
    lane = jax.lax.broadcasted_iota(jnp.int32, x.shape, 1)
    sh = 1
    while sh < S:
        r = pltpu.roll(x, sh, axis=1)
        x = x + jnp.where(lane >= sh, r, 0.0)
        sh *= 2
    return x


def _router_kernel(K, KP, x_ref, gw_ref, gb_ref, idxf_ref, idxi_ref, aux_ref):
    S = x_ref.shape[0]
    l = _dot(gw_ref[...], x_ref[...], ((1,), (1,))) + gb_ref[...]
    p = jax.nn.sigmoid(l)
    pm = jnp.sum(p, axis=1, keepdims=True) / S
    sp = (pm - 0.1) ** 2
    ent = -(p * jnp.log(p + 1e-10) + (1.0 - p) * jnp.log(1.0 - p + 1e-10))
    aux_ref[...] = 0.1 * sp + 0.01 * (jnp.sum(ent, axis=1, keepdims=True) / S)
    u = jax.lax.bitcast_convert_type(l, jnp.int32)
    key = jnp.where(u >= 0, u, u ^ jnp.int32(0x7FFFFFFF))
    MIN32 = jnp.int32(-(2 ** 31))
    tb = jnp.zeros((1, 1), jnp.int32)
    for b in range(31, -1, -1):
        bit = MIN32 if b == 31 else jnp.int32(1 << b)
        cand = tb | bit
        thr = cand ^ MIN32
        cnt = jnp.sum(jnp.where(key >= thr, 1.0, 0.0), axis=1, keepdims=True)
        tb = jnp.where(cnt >= K, cand, tb)
    vk = tb ^ MIN32
    gt = key > vk
    eq = key == vk
    C = jnp.sum(jnp.where(gt, 1.0, 0.0), axis=1, keepdims=True)
    eqf = jnp.where(eq, 1.0, 0.0)
    eqpos = _cumsum_lanes(eqf) - eqf
    sel = jnp.where(gt, 1.0, jnp.where(eq & (eqpos < (K - C)), 1.0, 0.0))
    pos = _cumsum_lanes(sel) - sel
    rows = jax.lax.broadcasted_iota(jnp.int32, (KP, 1), 0).astype(jnp.float32)
    lane = jax.lax.broadcasted_iota(jnp.int32, (KP, S), 1).astype(jnp.float32)
    onehot = jnp.where((sel > 0.5) & (pos == rows), 1.0, 0.0)
    idxf = jnp.sum(onehot * lane, axis=1, keepdims=True)
    idxf_ref[...] = idxf
    idxi_ref[...] = idxf.astype(jnp.int32)


def _kv_kernel(KVH, x_ref, anw_ref, wk_ref, wkr_ref, wv_ref, cos_ref, sin_ref,
               k_ref, v_ref):
    xn = _rms(x_ref[...], anw_ref[...])
    k0 = _dot(xn, wk_ref[...], ((1,), (1,)))
    kr = _dot(xn, wkr_ref[...], ((1,), (1,)))
    cos = jnp.concatenate([cos_ref[...]] * KVH, axis=1)
    sin = jnp.concatenate([sin_ref[...]] * KVH, axis=1)
    k = (k0 * cos + kr * sin).astype(jnp.bfloat16)
    v = _dot(xn, wv_ref[...], ((1,), (1,))).astype(jnp.bfloat16)
    for g in range(KVH):
        k_ref[g] = k[:, g * DH:(g + 1) * DH]
        v_ref[g] = v[:, g * DH:(g + 1) * DH]


def _make_sc_gather(D, CS, KP):
    info = plsc.get_sparse_core_info()
    NC, NS = info.num_cores, info.num_subcores
    NW = NC * NS
    BPW = KP // NW
    mesh = plsc.VectorSubcoreMesh(core_axis_name="c", subcore_axis_name="s")

    @functools.partial(
        pl.kernel, mesh=mesh,
        out_type=[jax.ShapeDtypeStruct((KP, D), jnp.float32),
                  jax.ShapeDtypeStruct((KP, CS), jnp.float32)],
        scratch_types=[pltpu.VMEM((BPW,), jnp.int32),
                       pltpu.VMEM((BPW, D), jnp.float32),
                       pltpu.VMEM((BPW, CS), jnp.float32),
                       pltpu.SemaphoreType.DMA],
    )
    def gather(x_hbm, cs_hbm, idx_hbm, xo_hbm, co_hbm, idx_v, xrows, crows, sem):
        wid = jax.lax.axis_index("s") * NC + jax.lax.axis_index("c")
        base = wid * BPW
        pltpu.sync_copy(idx_hbm.at[pl.ds(base, BPW)], idx_v)
        pltpu.async_copy(x_hbm.at[idx_v], xrows, sem).wait()
        pltpu.async_copy(cs_hbm.at[idx_v], crows, sem).wait()
        pltpu.sync_copy(xrows, xo_hbm.at[pl.ds(base, BPW)])
        pltpu.sync_copy(crows, co_hbm.at[pl.ds(base, BPW)])

    return gather


def _qsel_kernel(H, xs_ref, c_ref, anw_ref, wq_ref, wqr_ref, q_ref):
    xn = _rms(xs_ref[...], anw_ref[...])
    q0 = _dot(xn, wq_ref[...], ((1,), (1,)))
    qr = _dot(xn, wqr_ref[...], ((1,), (1,)))
    cos = jnp.concatenate([c_ref[:, :DH]] * H, axis=1)
    sin = jnp.concatenate([c_ref[:, DH:]] * H, axis=1)
    q = (q0 * cos + qr * sin).astype(jnp.bfloat16)
    for h in range(H):
        q_ref[h] = q[:, h * DH:(h + 1) * DH]


def _attn_kernel(QB, KT, K, scale, q_ref, k_ref, v_ref, idxf_ref, idxi_ref, o_ref):
    qb = pl.program_id(1)
    last_row = jnp.minimum((qb + 1) * QB - 1, K - 1)
    nt = idxi_ref[last_row, 0] // KT + 1
    qA = (q_ref[0].astype(jnp.float32) * scale).astype(jnp.bfloat16)
    qB = (q_ref[1].astype(jnp.float32) * scale).astype(jnp.bfloat16)
    idxc = idxf_ref[...]
    jc = jax.lax.broadcasted_iota(jnp.int32, (QB, KT), 1).astype(jnp.float32)

    def one(q, k, v, msk, m, den, acc):
        s = jnp.where(msk, _dot(q, k, ((1,), (1,))), NEG)
        mt = jnp.maximum(m, jnp.max(s, axis=1, keepdims=True))
        alpha = jnp.exp(m - mt)
        p = jnp.exp(s - mt)
        den = den * alpha + jnp.sum(p, axis=1, keepdims=True)
        acc = acc * alpha + _dot(p.astype(jnp.bfloat16), v, ((1,), (0,)))
        return mt, den, acc

    def body(t, carry):
        mA, dA, aA, mB, dB, aB = carry
        k = k_ref[0, pl.ds(t * KT, KT), :]
        v = v_ref[0, pl.ds(t * KT, KT), :]
        msk = idxc - (t * KT).astype(jnp.float32) >= jc
        mA, dA, aA = one(qA, k, v, msk, mA, dA, aA)
        mB, dB, aB = one(qB, k, v, msk, mB, dB, aB)
        return mA, dA, aA, mB, dB, aB

    m0 = jnp.full((QB, 1), NEG, jnp.float32)
    z1 = jnp.zeros((QB, 1), jnp.float32)
    z2 = jnp.zeros((QB, DH), jnp.float32)
    mA, dA, aA, mB, dB, aB = jax.lax.fori_loop(
        0, nt, body, (m0, z1, z2, m0, z1, z2))
    o_ref[0] = aA / dA
    o_ref[1] = aB / dB


def _oscatter_kernel(H, K, NSB, SB, x_ref, o_ref, wo_ref, idxi_ref,
                     h_ref, hf_ref, hl_ref, op_ref):
    o = jnp.concatenate([o_ref[h] for h in range(H)], axis=1)
    op_ref[...] = _dot(o, wo_ref[...], ((1,), (1,)))
    h_ref[...] = x_ref[...]

    def scat(i, _):
        r = idxi_ref[i, 0]
        h_ref[pl.ds(r, 1), :] = h_ref[pl.ds(r, 1), :] + op_ref[pl.ds(i, 1), :]
        return 0
    jax.lax.fori_loop(0, K, scat, 0)
    for s in range(NSB):
        hf_ref[s, 0:1, :] = h_ref[s * SB:s * SB + 1, :]
        hl_ref[s, 0:1, :] = h_ref[(s + 1) * SB - 1:(s + 1) * SB, :]


def _wcast_kernel(wg_ref, wu_ref, wd_ref, og_ref, ou_ref, od_ref):
    og_ref[...] = wg_ref[...].astype(jnp.float8_e4m3fn)
    ou_ref[...] = wu_ref[...].astype(jnp.float8_e4m3fn)
    od_ref[...] = wd_ref[...].astype(jnp.float8_e4m3fn)


def _mlp_kernel(NSB, rnw_ref, w0_ref, w1_ref, w2_ref, h_ref, hl_ref, hf_ref,
                wg_ref, wu_ref, wd_ref, y_ref):
    s = pl.program_id(0)
    rnw = rnw_ref[...]
    h = h_ref[...]
    hn = _rms(h, rnw)
    prev_row = jnp.where(s > 0, _rms(hl_ref[0], rnw), 0.0)
    next_row = jnp.where(s < NSB - 1, _rms(hf_ref[0], rnw), 0.0)
    prev = jnp.concatenate([prev_row, hn[:-1, :]], axis=0)
    nxt = jnp.concatenate([hn[1:, :], next_row], axis=0)
    hc = (w0_ref[...] * prev + w1_ref[...] * hn + w2_ref[...] * nxt).astype(jnp.float8_e4m3fn)
    g = _dot(hc, wg_ref[...], ((1,), (1,)))
    u = _dot(hc, wu_ref[...], ((1,), (1,)))
    a = (g * jax.nn.sigmoid(g) * u).astype(jnp.float8_e4m3fn)
    y_ref[...] = h + _dot(a, wd_ref[...], ((1,), (1,)))


def _rope_rows(W, nheads):
    Wh = W.reshape(nheads, DH, -1)
    return jnp.concatenate([-Wh[:, DH // 2:], Wh[:, :DH // 2]], axis=1).reshape(W.shape)


def kernel(x, Wq, Wk, Wv, Wo, attn_norm_w, gate_w, gate_b, log_temp,
           r_norm_w, conv_w, Wg, Wu, Wd):
    B, S, D = x.shape
    H = Wq.shape[0] // DH
    KVH = Wk.shape[0] // DH
    HID = Wg.shape[0]
    KQ = max(1, int(S * 0.1))
    KP = ((KQ + 127) // 128) * 128
    SB = min(512, S)
    NSB = S // SB
    GRP = H // KVH

    x2 = x.reshape(S, D)
    f32 = jnp.float32
    bf16 = jnp.bfloat16

    inv_freq = 1.0 / (10000.0 ** (jnp.arange(0, DH, 2, dtype=f32) / DH))
    fr = jnp.outer(jnp.arange(S, dtype=f32), inv_freq)
    emb = jnp.concatenate([fr, fr], axis=-1)
    cos64 = jnp.cos(emb)
    sin64 = jnp.sin(emb)
    cossin = jnp.concatenate([cos64, sin64], axis=1)
    WqR = _rope_rows(Wq, H)
    WkR = _rope_rows(Wk, KVH)
    anw = attn_norm_w.reshape(1, D)
    rnw = r_norm_w.reshape(1, D)
    gw = gate_w.reshape(1, D)
    gb = gate_b.reshape(1, 1)
    w0 = conv_w[:, 0, 0].reshape(1, D)
    w1 = conv_w[:, 0, 1].reshape(1, D)
    w2 = conv_w[:, 0, 2].reshape(1, D)

    full = lambda shp: pl.BlockSpec(shp, lambda *_: tuple(0 for _ in shp))
    smem = pl.BlockSpec(memory_space=pltpu.SMEM)

    idx_f, idx_i, aux = pl.pallas_call(
        functools.partial(_router_kernel, KQ, KP),
        out_shape=[jax.ShapeDtypeStruct((KP, 1), f32),
                   jax.ShapeDtypeStruct((KP, 1), jnp.int32),
                   jax.ShapeDtypeStruct((1, 1), f32)],
        in_specs=[full((S, D)), full((1, D)), full((1, 1))],
        out_specs=[full((KP, 1)), full((KP, 1)), full((1, 1))],
    )(x2, gw, gb)

    k4, v4 = pl.pallas_call(
        functools.partial(_kv_kernel, KVH),
        grid=(NSB,),
        out_shape=[jax.ShapeDtypeStruct((KVH, S, DH), bf16),
                   jax.ShapeDtypeStruct((KVH, S, DH), bf16)],
        in_specs=[pl.BlockSpec((SB, D), lambda s: (s, 0)),
                  full((1, D)),
                  full((KVH * DH, D)), full((KVH * DH, D)), full((KVH * DH, D)),
                  pl.BlockSpec((SB, DH), lambda s: (s, 0)),
                  pl.BlockSpec((SB, DH), lambda s: (s, 0))],
        out_specs=[pl.BlockSpec((KVH, SB, DH), lambda s: (0, s, 0)),
                   pl.BlockSpec((KVH, SB, DH), lambda s: (0, s, 0))],
    )(x2, anw, Wk, WkR, Wv, cos64, sin64)

    xsel, cssel = _make_sc_gather(D, 2 * DH, KP)(x2, cossin, idx_i.reshape(KP))

    q3 = pl.pallas_call(
        functools.partial(_qsel_kernel, H),
        out_shape=jax.ShapeDtypeStruct((H, KP, DH), bf16),
        in_specs=[full((KP, D)), full((KP, 2 * DH)), full((1, D)),
                  full((D, D)), full((D, D))],
        out_specs=full((H, KP, DH)),
    )(xsel, cssel, anw, Wq, WqR)

    QB = min(256, KP)
    KT = min(1024, S)
    NQB = KP // QB
    o3 = pl.pallas_call(
        functools.partial(_attn_kernel, QB, KT, KQ, 1.0 / (DH ** 0.5)),
        grid=(H // 2, NQB),
        out_shape=jax.ShapeDtypeStruct((H, KP, DH), f32),
        in_specs=[pl.BlockSpec((2, QB, DH), lambda p, qb: (p, qb, 0)),
                  pl.BlockSpec((1, S, DH), lambda p, qb: ((2 * p) // GRP, 0, 0)),
                  pl.BlockSpec((1, S, DH), lambda p, qb: ((2 * p) // GRP, 0, 0)),
                  pl.BlockSpec((QB, 1), lambda p, qb: (qb, 0)),
                  smem],
        out_specs=pl.BlockSpec((2, QB, DH), lambda p, qb: (p, qb, 0)),
    )(q3, k4, v4, idx_f, idx_i)

    SBM = min(256, S)
    NBM = S // SBM
    h, hfirst, hlast = pl.pallas_call(
        functools.partial(_oscatter_kernel, H, KQ, NBM, SBM),
        out_shape=[jax.ShapeDtypeStruct((S, D), f32),
                   jax.ShapeDtypeStruct((NBM, 1, D), f32),
                   jax.ShapeDtypeStruct((NBM, 1, D), f32)],
        in_specs=[full((S, D)), full((H, KP, DH)), full((D, D)), smem],
        out_specs=[full((S, D)), full((NBM, 1, D)), full((NBM, 1, D))],
        scratch_shapes=[pltpu.VMEM((KP, D), f32)],
    )(x2, o3, Wo, idx_i)

    NW8 = 8 if HID % 8 == 0 else 1
    Wg8, Wu8, Wd8 = pl.pallas_call(
        _wcast_kernel,
        grid=(NW8,),
        out_shape=[jax.ShapeDtypeStruct((HID, D), jnp.float8_e4m3fn),
                   jax.ShapeDtypeStruct((HID, D), jnp.float8_e4m3fn),
                   jax.ShapeDtypeStruct((D, HID), jnp.float8_e4m3fn)],
        in_specs=[pl.BlockSpec((HID // NW8, D), lambda s: (s, 0)),
                  pl.BlockSpec((HID // NW8, D), lambda s: (s, 0)),
                  pl.BlockSpec((D // NW8, HID), lambda s: (s, 0))],
        out_specs=[pl.BlockSpec((HID // NW8, D), lambda s: (s, 0)),
                   pl.BlockSpec((HID // NW8, D), lambda s: (s, 0)),
                   pl.BlockSpec((D // NW8, HID), lambda s: (s, 0))],
    )(Wg, Wu, Wd)

    SBL = min(512, S)
    NBL = S // SBL
    RL = SBL // SBM
    y = pl.pallas_call(
        functools.partial(_mlp_kernel, NBL),
        grid=(NBL,),
        out_shape=jax.ShapeDtypeStruct((S, D), f32),
        in_specs=[full((1, D)), full((1, D)), full((1, D)), full((1, D)),
                  pl.BlockSpec((SBL, D), lambda s: (s, 0)),
                  pl.BlockSpec((1, 1, D), lambda s: (jnp.maximum(s * RL - 1, 0), 0, 0)),
                  pl.BlockSpec((1, 1, D),
                               lambda s: (jnp.minimum((s + 1) * RL, NBM - 1), 0, 0)),
                  full((HID, D)), full((HID, D)), full((D, HID))],
        out_specs=pl.BlockSpec((SBL, D), lambda s: (s, 0)),
    )(rnw, w0, w1, w2, h, hlast, hfirst, Wg8, Wu8, Wd8)

    return y.reshape(B, S, D), aux[0, 0]


# speedup vs baseline: 1.0999x; 1.0273x over previous
import functools

import jax
import jax.numpy as jnp
from jax.experimental import pallas as pl
from jax.experimental.pallas import tpu as pltpu
from jax.experimental.pallas import tpu_sc as plsc

EPS = 1.1920929e-07
NEG = -1e30
DH = 64


def _rms(x, w):
    return x * jax.lax.rsqrt(jnp.mean(x * x, axis=-1, keepdims=True) + EPS) * w


def _dot(a, b, dims, out=jnp.float32):
    return jax.lax.dot_general(a, b, (dims, ((), ())),
                               preferred_element_type=out)


def _cumsum_lanes(x):
    S = x.shape[1]
    lane = jax.lax.broadcasted_iota(jnp.int32, x.shape, 1)
    sh = 1
    while sh < S:
        r = pltpu.roll(x, sh, axis=1)
        x = x + jnp.where(lane >= sh, r, 0.0)
        sh *= 2
    return x


def _router_kernel(K, KP, x_ref, gw_ref, gb_ref, idxf_ref, idxi_ref, aux_ref):
    S = x_ref.shape[0]
    l = _dot(gw_ref[...], x_ref[...], ((1,), (1,))) + gb_ref[...]
    p = jax.nn.sigmoid(l)
    pm = jnp.sum(p, axis=1, keepdims=True) / S
    sp = (pm - 0.1) ** 2
    ent = -(p * jnp.log(p + 1e-10) + (1.0 - p) * jnp.log(1.0 - p + 1e-10))
    aux_ref[...] = 0.1 * sp + 0.01 * (jnp.sum(ent, axis=1, keepdims=True) / S)
    u = jax.lax.bitcast_convert_type(l, jnp.int32)
    key = jnp.where(u >= 0, u, u ^ jnp.int32(0x7FFFFFFF))
    MIN32 = jnp.int32(-(2 ** 31))
    tb = jnp.zeros((1, 1), jnp.int32)
    for b in range(31, -1, -1):
        bit = MIN32 if b == 31 else jnp.int32(1 << b)
        cand = tb | bit
        thr = cand ^ MIN32
        cnt = jnp.sum(jnp.where(key >= thr, 1.0, 0.0), axis=1, keepdims=True)
        tb = jnp.where(cnt >= K, cand, tb)
    vk = tb ^ MIN32
    gt = key > vk
    eq = key == vk
    C = jnp.sum(jnp.where(gt, 1.0, 0.0), axis=1, keepdims=True)
    eqf = jnp.where(eq, 1.0, 0.0)
    eqpos = _cumsum_lanes(eqf) - eqf
    sel = jnp.where(gt, 1.0, jnp.where(eq & (eqpos < (K - C)), 1.0, 0.0))
    pos = _cumsum_lanes(sel) - sel
    rows = jax.lax.broadcasted_iota(jnp.int32, (KP, 1), 0).astype(jnp.float32)
    lane = jax.lax.broadcasted_iota(jnp.int32, (KP, S), 1).astype(jnp.float32)
    onehot = jnp.where((sel > 0.5) & (pos == rows), 1.0, 0.0)
    idxf = jnp.sum(onehot * lane, axis=1, keepdims=True)
    idxf_ref[...] = idxf
    idxi_ref[...] = idxf.astype(jnp.int32)


def _kv_kernel(KVH, x_ref, anw_ref, wk_ref, wkr_ref, wv_ref, cos_ref, sin_ref,
               k_ref, v_ref):
    xn = _rms(x_ref[...], anw_ref[...])
    k0 = _dot(xn, wk_ref[...], ((1,), (1,)))
    kr = _dot(xn, wkr_ref[...], ((1,), (1,)))
    cos = jnp.concatenate([cos_ref[...]] * KVH, axis=1)
    sin = jnp.concatenate([sin_ref[...]] * KVH, axis=1)
    k = (k0 * cos + kr * sin).astype(jnp.bfloat16)
    v = _dot(xn, wv_ref[...], ((1,), (1,))).astype(jnp.bfloat16)
    for g in range(KVH):
        k_ref[g] = k[:, g * DH:(g + 1) * DH]
        v_ref[g] = v[:, g * DH:(g + 1) * DH]


def _make_sc_gather(D, CS, KP):
    info = plsc.get_sparse_core_info()
    NC, NS = info.num_cores, info.num_subcores
    NW = NC * NS
    BPW = KP // NW
    mesh = plsc.VectorSubcoreMesh(core_axis_name="c", subcore_axis_name="s")

    @functools.partial(
        pl.kernel, mesh=mesh,
        out_type=[jax.ShapeDtypeStruct((KP, D), jnp.float32),
                  jax.ShapeDtypeStruct((KP, CS), jnp.float32)],
        scratch_types=[pltpu.VMEM((BPW,), jnp.int32),
                       pltpu.VMEM((BPW, D), jnp.float32),
                       pltpu.VMEM((BPW, CS), jnp.float32),
                       pltpu.SemaphoreType.DMA],
    )
    def gather(x_hbm, cs_hbm, idx_hbm, xo_hbm, co_hbm, idx_v, xrows, crows, sem):
        wid = jax.lax.axis_index("s") * NC + jax.lax.axis_index("c")
        base = wid * BPW
        pltpu.sync_copy(idx_hbm.at[pl.ds(base, BPW)], idx_v)
        pltpu.async_copy(x_hbm.at[idx_v], xrows, sem).wait()
        pltpu.async_copy(cs_hbm.at[idx_v], crows, sem).wait()
        pltpu.sync_copy(xrows, xo_hbm.at[pl.ds(base, BPW)])
        pltpu.sync_copy(crows, co_hbm.at[pl.ds(base, BPW)])

    return gather


def _qsel_kernel(H, xs_ref, c_ref, anw_ref, wq_ref, wqr_ref, q_ref):
    xn = _rms(xs_ref[...], anw_ref[...])
    q0 = _dot(xn, wq_ref[...], ((1,), (1,)))
    qr = _dot(xn, wqr_ref[...], ((1,), (1,)))
    cos = jnp.concatenate([c_ref[:, :DH]] * H, axis=1)
    sin = jnp.concatenate([c_ref[:, DH:]] * H, axis=1)
    q = (q0 * cos + qr * sin).astype(jnp.bfloat16)
    for h in range(H):
        q_ref[h] = q[:, h * DH:(h + 1) * DH]


def _attn_kernel(QB, KT, K, scale, q_ref, k_ref, v_ref, idxf_ref, idxi_ref, o_ref):
    qb = pl.program_id(1)
    last_row = jnp.minimum((qb + 1) * QB - 1, K - 1)
    nt = idxi_ref[last_row, 0] // KT + 1
    PH = q_ref.shape[0]
    qs = [(q_ref[i].astype(jnp.float32) * scale).astype(jnp.bfloat16)
          for i in range(PH)]
    idxc = idxf_ref[...]
    jc = jax.lax.broadcasted_iota(jnp.int32, (QB, KT), 1).astype(jnp.float32)

    def one(q, k, v, msk, m, den, acc):
        s = jnp.where(msk, _dot(q, k, ((1,), (1,))), NEG)
        mt = jnp.maximum(m, jnp.max(s, axis=1, keepdims=True))
        alpha = jnp.exp(m - mt)
        p = jnp.exp(s - mt)
        den = den * alpha + jnp.sum(p, axis=1, keepdims=True)
        acc = acc * alpha + _dot(p.astype(jnp.bfloat16), v, ((1,), (0,)))
        return mt, den, acc

    def body(t, carry):
        k = k_ref[0, pl.ds(t * KT, KT), :]
        v = v_ref[0, pl.ds(t * KT, KT), :]
        msk = idxc - (t * KT).astype(jnp.float32) >= jc
        return tuple(one(qs[i], k, v, msk, *carry[i]) for i in range(PH))

    m0 = jnp.full((QB, 1), NEG, jnp.float32)
    z1 = jnp.zeros((QB, 1), jnp.float32)
    z2 = jnp.zeros((QB, DH), jnp.float32)
    out = jax.lax.fori_loop(0, nt, body, tuple((m0, z1, z2) for _ in range(PH)))
    for i in range(PH):
        m, den, acc = out[i]
        o_ref[i] = acc / den


def _oscatter_kernel(H, K, NSB, SB, x_ref, o_ref, wo_ref, idxi_ref,
                     h_ref, hf_ref, hl_ref, op_ref):
    o = jnp.concatenate([o_ref[h] for h in range(H)], axis=1)
    op_ref[...] = _dot(o, wo_ref[...], ((1,), (1,)))
    h_ref[...] = x_ref[...]

    def scat(i, _):
        r = idxi_ref[i, 0]
        h_ref[pl.ds(r, 1), :] = h_ref[pl.ds(r, 1), :] + op_ref[pl.ds(i, 1), :]
        return 0
    jax.lax.fori_loop(0, K, scat, 0)
    for s in range(NSB):
        hf_ref[s, 0:1, :] = h_ref[s * SB:s * SB + 1, :]
        hl_ref[s, 0:1, :] = h_ref[(s + 1) * SB - 1:(s + 1) * SB, :]


def _wcast_kernel(wg_ref, wu_ref, wd_ref, og_ref, ou_ref, od_ref):
    og_ref[...] = wg_ref[...].astype(jnp.float8_e4m3fn)
    ou_ref[...] = wu_ref[...].astype(jnp.float8_e4m3fn)
    od_ref[...] = wd_ref[...].astype(jnp.float8_e4m3fn)


def _mlp_kernel(NSB, rnw_ref, w0_ref, w1_ref, w2_ref, h_ref, hl_ref, hf_ref,
                wg_ref, wu_ref, wd_ref, y_ref):
    s = pl.program_id(0)
    rnw = rnw_ref[...]
    h = h_ref[...]
    hn = _rms(h, rnw)
    prev_row = jnp.where(s > 0, _rms(hl_ref[0], rnw), 0.0)
    next_row = jnp.where(s < NSB - 1, _rms(hf_ref[0], rnw), 0.0)
    prev = jnp.concatenate([prev_row, hn[:-1, :]], axis=0)
    nxt = jnp.concatenate([hn[1:, :], next_row], axis=0)
    hc = (w0_ref[...] * prev + w1_ref[...] * hn + w2_ref[...] * nxt).astype(jnp.float8_e4m3fn)
    g = _dot(hc, wg_ref[...], ((1,), (1,)))
    u = _dot(hc, wu_ref[...], ((1,), (1,)))
    a = (g * jax.nn.sigmoid(g) * u).astype(jnp.float8_e4m3fn)
    y_ref[...] = h + _dot(a, wd_ref[...], ((1,), (1,)))


def _rope_rows(W, nheads):
    Wh = W.reshape(nheads, DH, -1)
    return jnp.concatenate([-Wh[:, DH // 2:], Wh[:, :DH // 2]], axis=1).reshape(W.shape)


def kernel(x, Wq, Wk, Wv, Wo, attn_norm_w, gate_w, gate_b, log_temp,
           r_norm_w, conv_w, Wg, Wu, Wd):
    B, S, D = x.shape
    H = Wq.shape[0] // DH
    KVH = Wk.shape[0] // DH
    HID = Wg.shape[0]
    KQ = max(1, int(S * 0.1))
    KP = ((KQ + 127) // 128) * 128
    SB = min(512, S)
    NSB = S // SB
    GRP = H // KVH

    x2 = x.reshape(S, D)
    f32 = jnp.float32
    bf16 = jnp.bfloat16

    inv_freq = 1.0 / (10000.0 ** (jnp.arange(0, DH, 2, dtype=f32) / DH))
    fr = jnp.outer(jnp.arange(S, dtype=f32), inv_freq)
    emb = jnp.concatenate([fr, fr], axis=-1)
    cos64 = jnp.cos(emb)
    sin64 = jnp.sin(emb)
    cossin = jnp.concatenate([cos64, sin64], axis=1)
    WqR = _rope_rows(Wq, H)
    WkR = _rope_rows(Wk, KVH)
    anw = attn_norm_w.reshape(1, D)
    rnw = r_norm_w.reshape(1, D)
    gw = gate_w.reshape(1, D)
    gb = gate_b.reshape(1, 1)
    w0 = conv_w[:, 0, 0].reshape(1, D)
    w1 = conv_w[:, 0, 1].reshape(1, D)
    w2 = conv_w[:, 0, 2].reshape(1, D)

    full = lambda shp: pl.BlockSpec(shp, lambda *_: tuple(0 for _ in shp))
    smem = pl.BlockSpec(memory_space=pltpu.SMEM)

    idx_f, idx_i, aux = pl.pallas_call(
        functools.partial(_router_kernel, KQ, KP),
        out_shape=[jax.ShapeDtypeStruct((KP, 1), f32),
                   jax.ShapeDtypeStruct((KP, 1), jnp.int32),
                   jax.ShapeDtypeStruct((1, 1), f32)],
        in_specs=[full((S, D)), full((1, D)), full((1, 1))],
        out_specs=[full((KP, 1)), full((KP, 1)), full((1, 1))],
    )(x2, gw, gb)

    k4, v4 = pl.pallas_call(
        functools.partial(_kv_kernel, KVH),
        grid=(NSB,),
        out_shape=[jax.ShapeDtypeStruct((KVH, S, DH), bf16),
                   jax.ShapeDtypeStruct((KVH, S, DH), bf16)],
        in_specs=[pl.BlockSpec((SB, D), lambda s: (s, 0)),
                  full((1, D)),
                  full((KVH * DH, D)), full((KVH * DH, D)), full((KVH * DH, D)),
                  pl.BlockSpec((SB, DH), lambda s: (s, 0)),
                  pl.BlockSpec((SB, DH), lambda s: (s, 0))],
        out_specs=[pl.BlockSpec((KVH, SB, DH), lambda s: (0, s, 0)),
                   pl.BlockSpec((KVH, SB, DH), lambda s: (0, s, 0))],
    )(x2, anw, Wk, WkR, Wv, cos64, sin64)

    xsel, cssel = _make_sc_gather(D, 2 * DH, KP)(x2, cossin, idx_i.reshape(KP))

    q3 = pl.pallas_call(
        functools.partial(_qsel_kernel, H),
        out_shape=jax.ShapeDtypeStruct((H, KP, DH), bf16),
        in_specs=[full((KP, D)), full((KP, 2 * DH)), full((1, D)),
                  full((D, D)), full((D, D))],
        out_specs=full((H, KP, DH)),
    )(xsel, cssel, anw, Wq, WqR)

    QB = min(256, KP)
    KT = min(1024, S)
    NQB = KP // QB
    PH = min(4, GRP) if H % min(4, GRP) == 0 else 1
    o3 = pl.pallas_call(
        functools.partial(_attn_kernel, QB, KT, KQ, 1.0 / (DH ** 0.5)),
        grid=(H // PH, NQB),
        out_shape=jax.ShapeDtypeStruct((H, KP, DH), f32),
        in_specs=[pl.BlockSpec((PH, QB, DH), lambda p, qb: (p, qb, 0)),
                  pl.BlockSpec((1, S, DH), lambda p, qb: ((PH * p) // GRP, 0, 0)),
                  pl.BlockSpec((1, S, DH), lambda p, qb: ((PH * p) // GRP, 0, 0)),
                  pl.BlockSpec((QB, 1), lambda p, qb: (qb, 0)),
                  smem],
        out_specs=pl.BlockSpec((PH, QB, DH), lambda p, qb: (p, qb, 0)),
    )(q3, k4, v4, idx_f, idx_i)

    SBM = min(256, S)
    NBM = S // SBM
    h, hfirst, hlast = pl.pallas_call(
        functools.partial(_oscatter_kernel, H, KQ, NBM, SBM),
        out_shape=[jax.ShapeDtypeStruct((S, D), f32),
                   jax.ShapeDtypeStruct((NBM, 1, D), f32),
                   jax.ShapeDtypeStruct((NBM, 1, D), f32)],
        in_specs=[full((S, D)), full((H, KP, DH)), full((D, D)), smem],
        out_specs=[full((S, D)), full((NBM, 1, D)), full((NBM, 1, D))],
        scratch_shapes=[pltpu.VMEM((KP, D), f32)],
    )(x2, o3, Wo, idx_i)

    NW8 = 8 if HID % 8 == 0 else 1
    Wg8, Wu8, Wd8 = pl.pallas_call(
        _wcast_kernel,
        grid=(NW8,),
        out_shape=[jax.ShapeDtypeStruct((HID, D), jnp.float8_e4m3fn),
                   jax.ShapeDtypeStruct((HID, D), jnp.float8_e4m3fn),
                   jax.ShapeDtypeStruct((D, HID), jnp.float8_e4m3fn)],
        in_specs=[pl.BlockSpec((HID // NW8, D), lambda s: (s, 0)),
                  pl.BlockSpec((HID // NW8, D), lambda s: (s, 0)),
                  pl.BlockSpec((D // NW8, HID), lambda s: (s, 0))],
        out_specs=[pl.BlockSpec((HID // NW8, D), lambda s: (s, 0)),
                   pl.BlockSpec((HID // NW8, D), lambda s: (s, 0)),
                   pl.BlockSpec((D // NW8, HID), lambda s: (s, 0))],
    )(Wg, Wu, Wd)

    SBL = min(512, S)
    NBL = S // SBL
    RL = SBL // SBM
    y = pl.pallas_call(
        functools.partial(_mlp_kernel, NBL),
        grid=(NBL,),
        out_shape=jax.ShapeDtypeStruct((S, D), f32),
        in_specs=[full((1, D)), full((1, D)), full((1, D)), full((1, D)),
                  pl.BlockSpec((SBL, D), lambda s: (s, 0)),
                  pl.BlockSpec((1, 1, D), lambda s: (jnp.maximum(s * RL - 1, 0), 0, 0)),
                  pl.BlockSpec((1, 1, D),
                               lambda s: (jnp.minimum((s + 1) * RL, NBM - 1), 0, 0)),
                  full((HID, D)), full((HID, D)), full((D, HID))],
        out_specs=pl.BlockSpec((SBL, D), lambda s: (s, 0)),
    )(rnw, w0, w1, w2, h, hlast, hfirst, Wg8, Wu8, Wd8)

    return y.reshape(B, S, D), aux[0, 0]
